# initial kernel scaffold (unmeasured)
import jax
import jax.numpy as jnp
from jax import lax
from jax.experimental import pallas as pl
from jax.experimental.pallas import tpu as pltpu


def _exchange_body(p_ref, out_ref, send_sem, recv_sem):
    my_x = lax.axis_index("x")
    my_y = lax.axis_index("y")
    my_z = lax.axis_index("z")
    peer = (1 - my_x, my_y, my_z)

    barrier = pltpu.get_barrier_semaphore()
    pl.semaphore_signal(
        barrier, inc=1, device_id=peer, device_id_type=pl.DeviceIdType.MESH
    )
    pl.semaphore_wait(barrier, 1)

    rdma = pltpu.make_async_remote_copy(
        src_ref=p_ref,
        dst_ref=out_ref,
        send_sem=send_sem,
        recv_sem=recv_sem,
        device_id=peer,
        device_id_type=pl.DeviceIdType.MESH,
    )
    rdma.start()
    rdma.wait()


def _exchange(p_bf):
    m, d = p_bf.shape
    return pl.pallas_call(
        _exchange_body,
        out_shape=jax.ShapeDtypeStruct((m, d), p_bf.dtype),
        in_specs=[pl.BlockSpec(memory_space=pltpu.ANY)],
        out_specs=pl.BlockSpec(memory_space=pltpu.ANY),
        scratch_shapes=[pltpu.SemaphoreType.DMA, pltpu.SemaphoreType.DMA],
        compiler_params=pltpu.CompilerParams(collective_id=0),
    )(p_bf)


_BLOCK_M = 512


def _ln_body(p_ref, q_ref, r_ref, g_ref, o_ref):
    y = (
        p_ref[...].astype(jnp.float32)
        + q_ref[...].astype(jnp.float32)
        + r_ref[...]
    )
    ms = jnp.mean(y * y, axis=-1, keepdims=True)
    o_ref[...] = y * lax.rsqrt(ms + 1e-6) * g_ref[...]


def _ln(p_bf, q_bf, resid, gamma2d):
    m, d = resid.shape
    return pl.pallas_call(
        _ln_body,
        grid=(m // _BLOCK_M,),
        in_specs=[
            pl.BlockSpec((_BLOCK_M, d), lambda i: (i, 0)),
            pl.BlockSpec((_BLOCK_M, d), lambda i: (i, 0)),
            pl.BlockSpec((_BLOCK_M, d), lambda i: (i, 0)),
            pl.BlockSpec((1, d), lambda i: (0, 0)),
        ],
        out_specs=pl.BlockSpec((_BLOCK_M, d), lambda i: (i, 0)),
        out_shape=jax.ShapeDtypeStruct((m, d), jnp.float32),
    )(p_bf, q_bf, resid, gamma2d)


def kernel(partial, resid, gamma):
    p_bf = partial[0].astype(jnp.bfloat16)
    q_bf = _exchange(p_bf)
    return _ln(p_bf, q_bf, resid, gamma.reshape(1, -1))


# baseline (device time: 463293 ns/iter reference)
import jax
import jax.numpy as jnp
from jax import lax
from jax.experimental import pallas as pl
from jax.experimental.pallas import tpu as pltpu


def _exchange_body(p_ref, out_ref, send_sem, recv_sem):
    my_x = lax.axis_index("x")
    my_y = lax.axis_index("y")
    my_z = lax.axis_index("z")
    peer = (1 - my_x, my_y, my_z)

    barrier = pltpu.get_barrier_semaphore()
    pl.semaphore_signal(
        barrier, inc=1, device_id=peer, device_id_type=pl.DeviceIdType.MESH
    )
    pl.semaphore_wait(barrier, 1)

    rdma = pltpu.make_async_remote_copy(
        src_ref=p_ref,
        dst_ref=out_ref,
        send_sem=send_sem,
        recv_sem=recv_sem,
        device_id=peer,
        device_id_type=pl.DeviceIdType.MESH,
    )
    rdma.start()
    rdma.wait()


def _exchange(p_bf):
    m, d = p_bf.shape
    return pl.pallas_call(
        _exchange_body,
        out_shape=jax.ShapeDtypeStruct((m, d), p_bf.dtype),
        in_specs=[pl.BlockSpec(memory_space=pl.ANY)],
        out_specs=pl.BlockSpec(memory_space=pl.ANY),
        scratch_shapes=[pltpu.SemaphoreType.DMA, pltpu.SemaphoreType.DMA],
        compiler_params=pltpu.CompilerParams(collective_id=0),
    )(p_bf)


_BLOCK_M = 512


def _ln_body(p_ref, q_ref, r_ref, g_ref, o_ref):
    y = (
        p_ref[...].astype(jnp.float32)
        + q_ref[...].astype(jnp.float32)
        + r_ref[...]
    )
    ms = jnp.mean(y * y, axis=-1, keepdims=True)
    o_ref[...] = y * lax.rsqrt(ms + 1e-6) * g_ref[...]


def _ln(p_bf, q_bf, resid, gamma2d):
    m, d = resid.shape
    return pl.pallas_call(
        _ln_body,
        grid=(m // _BLOCK_M,),
        in_specs=[
            pl.BlockSpec((_BLOCK_M, d), lambda i: (i, 0)),
            pl.BlockSpec((_BLOCK_M, d), lambda i: (i, 0)),
            pl.BlockSpec((_BLOCK_M, d), lambda i: (i, 0)),
            pl.BlockSpec((1, d), lambda i: (0, 0)),
        ],
        out_specs=pl.BlockSpec((_BLOCK_M, d), lambda i: (i, 0)),
        out_shape=jax.ShapeDtypeStruct((m, d), jnp.float32),
    )(p_bf, q_bf, resid, gamma2d)


def kernel(partial, resid, gamma):
    p_bf = partial[0].astype(jnp.bfloat16)
    q_bf = _exchange(p_bf)
    return _ln(p_bf, q_bf, resid, gamma.reshape(1, -1))


# device time: 249517 ns/iter; 1.8568x vs baseline; 1.8568x over previous
import jax
import jax.numpy as jnp
from jax import lax
from jax.experimental import pallas as pl
from jax.experimental.pallas import tpu as pltpu

M = 8192
D = 2048
BLK = 2048
N_CH = 8
CH = BLK // N_CH

_MESH_ID = pl.DeviceIdType.MESH


def _body(
    partial_ref,
    resid_ref,
    gamma_ref,
    out_ref,
    sendx,
    recvx,
    rbf,
    recvy,
    recvzo,
    recvzf,
    pstage,
    rstage,
    ostage,
    cstage,
    x_send,
    x_recv,
    y_send,
    y_recv,
    zo_send,
    zo_recv,
    zf_send,
    zf_recv,
    pin_sem,
    rin_sem,
    oout_sem,
    cout_sem,
    rv,
):
    my_x = lax.axis_index("x")
    my_y = lax.axis_index("y")
    my_z = lax.axis_index("z")
    xp = (1 - my_x, my_y, my_z)
    yp = (my_x, 1 - my_y, my_z)
    zp = (my_x, my_y, 1 - my_z)
    b = 2 * my_y + my_z
    b_y = 2 * (1 - my_y) + my_z
    b_z = 2 * my_y + (1 - my_z)
    b_d = 2 * (1 - my_y) + (1 - my_z)

    barrier = pltpu.get_barrier_semaphore()
    for nbr in (xp, yp, zp):
        pl.semaphore_signal(barrier, inc=1, device_id=nbr, device_id_type=_MESH_ID)
    pl.semaphore_wait(barrier, 3)
    for k, nbr in enumerate((xp, yp, zp)):
        pl.semaphore_signal(rv.at[k], inc=1, device_id=nbr, device_id_type=_MESH_ID)
    for k in range(3):
        pl.semaphore_wait(rv.at[k], 1)

    row0 = b * BLK

    x_rdmas = []
    for c in range(N_CH):
        cp = pltpu.make_async_copy(
            partial_ref.at[0, pl.ds(row0 + c * CH, CH), :], pstage, pin_sem
        )
        cp.start()
        cp.wait()
        sendx[c] = pstage[...].astype(jnp.bfloat16)
        rdma = pltpu.make_async_remote_copy(
            src_ref=sendx.at[c],
            dst_ref=recvx.at[c],
            send_sem=x_send.at[c],
            recv_sem=x_recv.at[c],
            device_id=xp,
            device_id_type=_MESH_ID,
        )
        rdma.start()
        x_rdmas.append(rdma)

    y_rdmas = []
    zo_rdmas = []
    o_copies = [None, None]
    for c in range(N_CH):
        x_rdmas[c].wait_recv()
        cpr = pltpu.make_async_copy(
            resid_ref.at[pl.ds(row0 + c * CH, CH), :], rstage, rin_sem
        )
        cpr.start()
        cpr.wait()
        s = c % 2
        if o_copies[s] is not None:
            o_copies[s].wait()
        yv = (
            sendx[c].astype(jnp.float32)
            + recvx[c].astype(jnp.float32)
            + rstage[...]
        )
        ms = jnp.mean(yv * yv, axis=-1, keepdims=True)
        ov = yv * lax.rsqrt(ms + 1e-6) * gamma_ref[...]
        ostage[s] = ov
        rbf[c] = ov.astype(jnp.bfloat16)
        oc = pltpu.make_async_copy(
            ostage.at[s], out_ref.at[pl.ds(row0 + c * CH, CH), :], oout_sem.at[s]
        )
        oc.start()
        o_copies[s] = oc
        for dst, ss, rs, lst, peer in (
            (recvy, y_send, y_recv, y_rdmas, yp),
            (recvzo, zo_send, zo_recv, zo_rdmas, zp),
        ):
            rdma = pltpu.make_async_remote_copy(
                src_ref=rbf.at[c],
                dst_ref=dst.at[c],
                send_sem=ss.at[c],
                recv_sem=rs.at[c],
                device_id=peer,
                device_id_type=_MESH_ID,
            )
            rdma.start()
            lst.append(rdma)

    c_copies = [None, None]

    def consume(buf, c, dst_row0):
        s = c % 2
        if c_copies[s] is not None:
            c_copies[s].wait()
        cstage[s] = buf[c].astype(jnp.float32)
        cc = pltpu.make_async_copy(
            cstage.at[s],
            out_ref.at[pl.ds(dst_row0 + c * CH, CH), :],
            cout_sem.at[s],
        )
        cc.start()
        c_copies[s] = cc

    zf_rdmas = []
    for c in range(N_CH):
        y_rdmas[c].wait_recv()
        zf = pltpu.make_async_remote_copy(
            src_ref=recvy.at[c],
            dst_ref=recvzf.at[c],
            send_sem=zf_send.at[c],
            recv_sem=zf_recv.at[c],
            device_id=zp,
            device_id_type=_MESH_ID,
        )
        zf.start()
        zf_rdmas.append(zf)
        consume(recvy, c, b_y * BLK)
    for c in range(N_CH):
        zo_rdmas[c].wait_recv()
        consume(recvzo, c, b_z * BLK)
    for c in range(N_CH):
        zf_rdmas[c].wait_recv()
        consume(recvzf, c, b_d * BLK)

    for r in x_rdmas + y_rdmas + zo_rdmas + zf_rdmas:
        r.wait_send()
    for oc in o_copies + c_copies:
        if oc is not None:
            oc.wait()


def kernel(partial, resid, gamma):
    chunk = (N_CH, CH, D)
    return pl.pallas_call(
        _body,
        out_shape=jax.ShapeDtypeStruct((M, D), jnp.float32),
        in_specs=[
            pl.BlockSpec(memory_space=pl.ANY),
            pl.BlockSpec(memory_space=pl.ANY),
            pl.BlockSpec(memory_space=pltpu.MemorySpace.VMEM),
        ],
        out_specs=pl.BlockSpec(memory_space=pl.ANY),
        scratch_shapes=[
            pltpu.VMEM(chunk, jnp.bfloat16),
            pltpu.VMEM(chunk, jnp.bfloat16),
            pltpu.VMEM(chunk, jnp.bfloat16),
            pltpu.VMEM(chunk, jnp.bfloat16),
            pltpu.VMEM(chunk, jnp.bfloat16),
            pltpu.VMEM(chunk, jnp.bfloat16),
            pltpu.VMEM((CH, D), jnp.float32),
            pltpu.VMEM((CH, D), jnp.float32),
            pltpu.VMEM((2, CH, D), jnp.float32),
            pltpu.VMEM((2, CH, D), jnp.float32),
            pltpu.SemaphoreType.DMA((N_CH,)),
            pltpu.SemaphoreType.DMA((N_CH,)),
            pltpu.SemaphoreType.DMA((N_CH,)),
            pltpu.SemaphoreType.DMA((N_CH,)),
            pltpu.SemaphoreType.DMA((N_CH,)),
            pltpu.SemaphoreType.DMA((N_CH,)),
            pltpu.SemaphoreType.DMA((N_CH,)),
            pltpu.SemaphoreType.DMA((N_CH,)),
            pltpu.SemaphoreType.DMA,
            pltpu.SemaphoreType.DMA,
            pltpu.SemaphoreType.DMA((2,)),
            pltpu.SemaphoreType.DMA((2,)),
            pltpu.SemaphoreType.REGULAR((3,)),
        ],
        compiler_params=pltpu.CompilerParams(
            collective_id=0, vmem_limit_bytes=100 * 1024 * 1024
        ),
    )(partial, resid, gamma.reshape(1, -1))


# device time: 248963 ns/iter; 1.8609x vs baseline; 1.0022x over previous
import jax
import jax.numpy as jnp
from jax import lax
from jax.experimental import pallas as pl
from jax.experimental.pallas import tpu as pltpu

M = 8192
D = 2048
BLK = 2048
N_CH = 8
CH = BLK // N_CH

_MESH_ID = pl.DeviceIdType.MESH


def _body(
    partial_ref,
    resid_ref,
    gamma_ref,
    out_ref,
    sendx,
    recvx,
    recvy,
    recvzo,
    recvzf,
    pstage,
    rstage,
    ostage,
    cstage,
    x_send,
    x_recv,
    y_send,
    y_recv,
    zo_send,
    zo_recv,
    zf_send,
    zf_recv,
    pin_sem,
    rin_sem,
    oout_sem,
    cout_sem,
    rv,
):
    my_x = lax.axis_index("x")
    my_y = lax.axis_index("y")
    my_z = lax.axis_index("z")
    xp = (1 - my_x, my_y, my_z)
    yp = (my_x, 1 - my_y, my_z)
    zp = (my_x, my_y, 1 - my_z)
    b = 2 * my_y + my_z
    b_y = 2 * (1 - my_y) + my_z
    b_z = 2 * my_y + (1 - my_z)
    b_d = 2 * (1 - my_y) + (1 - my_z)

    barrier = pltpu.get_barrier_semaphore()
    for nbr in (xp, yp, zp):
        pl.semaphore_signal(barrier, inc=1, device_id=nbr, device_id_type=_MESH_ID)
    pl.semaphore_wait(barrier, 3)
    for k, nbr in enumerate((xp, yp, zp)):
        pl.semaphore_signal(rv.at[k], inc=1, device_id=nbr, device_id_type=_MESH_ID)
    for k in range(3):
        pl.semaphore_wait(rv.at[k], 1)

    row0 = b * BLK

    def stage_partial(c):
        cp = pltpu.make_async_copy(
            partial_ref.at[0, pl.ds(row0 + c * CH, CH), :],
            pstage.at[c % 2],
            pin_sem.at[c % 2],
        )
        cp.start()
        return cp

    def stage_resid(c):
        cp = pltpu.make_async_copy(
            resid_ref.at[pl.ds(row0 + c * CH, CH), :],
            rstage.at[c % 2],
            rin_sem.at[c % 2],
        )
        cp.start()
        return cp

    p_copies = [stage_partial(0), stage_partial(1)]
    r_copies = [stage_resid(0), stage_resid(1)]

    x_rdmas = []
    for c in range(N_CH):
        p_copies[c].wait()
        sendx[c] = pstage[c % 2].astype(jnp.bfloat16)
        if c + 2 < N_CH:
            p_copies.append(stage_partial(c + 2))
        rdma = pltpu.make_async_remote_copy(
            src_ref=sendx.at[c],
            dst_ref=recvx.at[c],
            send_sem=x_send.at[c],
            recv_sem=x_recv.at[c],
            device_id=xp,
            device_id_type=_MESH_ID,
        )
        rdma.start()
        x_rdmas.append(rdma)

    y_rdmas = []
    zo_rdmas = []
    o_copies = [None, None]
    for c in range(N_CH):
        x_rdmas[c].wait_recv()
        r_copies[c].wait()
        s = c % 2
        if o_copies[s] is not None:
            o_copies[s].wait()
        yv = (
            sendx[c].astype(jnp.float32)
            + recvx[c].astype(jnp.float32)
            + rstage[s]
        )
        ms = jnp.mean(yv * yv, axis=-1, keepdims=True)
        ov = yv * lax.rsqrt(ms + 1e-6) * gamma_ref[...]
        ostage[s] = ov
        x_rdmas[c].wait_send()
        sendx[c] = ov.astype(jnp.bfloat16)
        if c + 2 < N_CH:
            r_copies.append(stage_resid(c + 2))
        oc = pltpu.make_async_copy(
            ostage.at[s], out_ref.at[pl.ds(row0 + c * CH, CH), :], oout_sem.at[s]
        )
        oc.start()
        o_copies[s] = oc
        for dst, ss, rs, lst, peer in (
            (recvy, y_send, y_recv, y_rdmas, yp),
            (recvzo, zo_send, zo_recv, zo_rdmas, zp),
        ):
            rdma = pltpu.make_async_remote_copy(
                src_ref=sendx.at[c],
                dst_ref=dst.at[c],
                send_sem=ss.at[c],
                recv_sem=rs.at[c],
                device_id=peer,
                device_id_type=_MESH_ID,
            )
            rdma.start()
            lst.append(rdma)

    c_copies = [None, None]

    def consume(buf, c, dst_row0):
        s = c % 2
        if c_copies[s] is not None:
            c_copies[s].wait()
        cstage[s] = buf[c].astype(jnp.float32)
        cc = pltpu.make_async_copy(
            cstage.at[s],
            out_ref.at[pl.ds(dst_row0 + c * CH, CH), :],
            cout_sem.at[s],
        )
        cc.start()
        c_copies[s] = cc

    zf_rdmas = []
    for c in range(N_CH):
        y_rdmas[c].wait_recv()
        zf = pltpu.make_async_remote_copy(
            src_ref=recvy.at[c],
            dst_ref=recvzf.at[c],
            send_sem=zf_send.at[c],
            recv_sem=zf_recv.at[c],
            device_id=zp,
            device_id_type=_MESH_ID,
        )
        zf.start()
        zf_rdmas.append(zf)
        consume(recvy, c, b_y * BLK)
    for c in range(N_CH):
        zo_rdmas[c].wait_recv()
        consume(recvzo, c, b_z * BLK)
    for c in range(N_CH):
        zf_rdmas[c].wait_recv()
        consume(recvzf, c, b_d * BLK)

    for r in y_rdmas + zo_rdmas + zf_rdmas:
        r.wait_send()
    for oc in o_copies + c_copies:
        if oc is not None:
            oc.wait()


def kernel(partial, resid, gamma):
    chunk = (N_CH, CH, D)
    return pl.pallas_call(
        _body,
        out_shape=jax.ShapeDtypeStruct((M, D), jnp.float32),
        in_specs=[
            pl.BlockSpec(memory_space=pl.ANY),
            pl.BlockSpec(memory_space=pl.ANY),
            pl.BlockSpec(memory_space=pltpu.MemorySpace.VMEM),
        ],
        out_specs=pl.BlockSpec(memory_space=pl.ANY),
        scratch_shapes=[
            pltpu.VMEM(chunk, jnp.bfloat16),
            pltpu.VMEM(chunk, jnp.bfloat16),
            pltpu.VMEM(chunk, jnp.bfloat16),
            pltpu.VMEM(chunk, jnp.bfloat16),
            pltpu.VMEM(chunk, jnp.bfloat16),
            pltpu.VMEM((2, CH, D), jnp.float32),
            pltpu.VMEM((2, CH, D), jnp.float32),
            pltpu.VMEM((2, CH, D), jnp.float32),
            pltpu.VMEM((2, CH, D), jnp.float32),
            pltpu.SemaphoreType.DMA((N_CH,)),
            pltpu.SemaphoreType.DMA((N_CH,)),
            pltpu.SemaphoreType.DMA((N_CH,)),
            pltpu.SemaphoreType.DMA((N_CH,)),
            pltpu.SemaphoreType.DMA((N_CH,)),
            pltpu.SemaphoreType.DMA((N_CH,)),
            pltpu.SemaphoreType.DMA((N_CH,)),
            pltpu.SemaphoreType.DMA((N_CH,)),
            pltpu.SemaphoreType.DMA((2,)),
            pltpu.SemaphoreType.DMA((2,)),
            pltpu.SemaphoreType.DMA((2,)),
            pltpu.SemaphoreType.DMA((2,)),
            pltpu.SemaphoreType.REGULAR((3,)),
        ],
        compiler_params=pltpu.CompilerParams(
            collective_id=0, vmem_limit_bytes=100 * 1024 * 1024
        ),
    )(partial, resid, gamma.reshape(1, -1))


# device time: 242789 ns/iter; 1.9082x vs baseline; 1.0254x over previous
import jax
import jax.numpy as jnp
from jax import lax
from jax.experimental import pallas as pl
from jax.experimental.pallas import tpu as pltpu

M = 8192
D = 2048
BLK = 2048
N_CH = 8
CH = BLK // N_CH
HH = CH // 2

_MESH_ID = pl.DeviceIdType.MESH


def _body(
    partial_ref,
    resid_ref,
    gamma_ref,
    out_ref,
    sxbuf,
    recvx,
    rblk,
    recvy,
    recvzo,
    recvzf,
    recvyf,
    pxstage,
    pmstage,
    rstage,
    ostage,
    cstage,
    phstage,
    x1_send,
    x1_recv,
    x2_send,
    x2_recv,
    y_send,
    y_recv,
    zo_send,
    zo_recv,
    zf_send,
    zf_recv,
    yf_send,
    yf_recv,
    pxin_sem,
    pmin_sem,
    rin_sem,
    oout_sem,
    cout_sem,
    ph_sem,
    rv,
):
    my_x = lax.axis_index("x")
    my_y = lax.axis_index("y")
    my_z = lax.axis_index("z")
    xp = (1 - my_x, my_y, my_z)
    yp = (my_x, 1 - my_y, my_z)
    zp = (my_x, my_y, 1 - my_z)
    b = 2 * my_y + my_z
    b_y = 2 * (1 - my_y) + my_z
    b_z = 2 * my_y + (1 - my_z)
    b_d = 2 * (1 - my_y) + (1 - my_z)

    barrier = pltpu.get_barrier_semaphore()
    for nbr in (xp, yp, zp):
        pl.semaphore_signal(barrier, inc=1, device_id=nbr, device_id_type=_MESH_ID)
    pl.semaphore_wait(barrier, 3)
    for k, nbr in enumerate((xp, yp, zp)):
        pl.semaphore_signal(rv.at[k], inc=1, device_id=nbr, device_id_type=_MESH_ID)
    for k in range(3):
        pl.semaphore_wait(rv.at[k], 1)

    row0 = b * BLK
    hx = my_x * HH

    def stage(src_rows, dst, sem):
        cp = pltpu.make_async_copy(
            resid_ref.at[pl.ds(src_rows, HH), :], dst, sem
        )
        cp.start()
        return cp

    def stage_partial(src_rows, dst, sem):
        cp = pltpu.make_async_copy(
            partial_ref.at[0, pl.ds(src_rows, HH), :], dst, sem
        )
        cp.start()
        return cp

    hpx = (1 - my_x) * HH

    px_copies = [
        stage_partial(row0 + c * CH + hpx, pxstage.at[c % 2], pxin_sem.at[c % 2])
        for c in range(2)
    ]
    pm_copies = [
        stage_partial(row0 + c * CH + hx, pmstage.at[c % 2], pmin_sem.at[c % 2])
        for c in range(2)
    ]
    r_copies = [
        stage(row0 + c * CH + hx, rstage.at[c % 2], rin_sem.at[c % 2])
        for c in range(2)
    ]

    x1_rdmas = []
    for c in range(N_CH):
        px_copies[c].wait()
        sxbuf[c] = pxstage[c % 2].astype(jnp.bfloat16)
        if c + 2 < N_CH:
            px_copies.append(
                stage_partial(
                    row0 + (c + 2) * CH + hpx,
                    pxstage.at[c % 2],
                    pxin_sem.at[c % 2],
                )
            )
        rdma = pltpu.make_async_remote_copy(
            src_ref=sxbuf.at[c],
            dst_ref=recvx.at[c],
            send_sem=x1_send.at[c],
            recv_sem=x1_recv.at[c],
            device_id=xp,
            device_id_type=_MESH_ID,
        )
        rdma.start()
        x1_rdmas.append(rdma)

    x2_rdmas = []
    y_rdmas = []
    zo_rdmas = []
    o_copies = [None, None]
    ph_copies = [None, None]

    def consume_peer_half(c):
        s = c % 2
        if ph_copies[s] is not None:
            ph_copies[s].wait()

        @pl.when(my_x == 0)
        def _():
            phstage[s] = rblk[c, HH:CH].astype(jnp.float32)

        @pl.when(my_x == 1)
        def _():
            phstage[s] = rblk[c, 0:HH].astype(jnp.float32)

        cc = pltpu.make_async_copy(
            phstage.at[s],
            out_ref.at[pl.ds(row0 + c * CH + hpx, HH), :],
            ph_sem.at[s],
        )
        cc.start()
        ph_copies[s] = cc

    def send_gather(c):
        out = []
        for dst, ss, rs, peer in (
            (recvy, y_send, y_recv, yp),
            (recvzo, zo_send, zo_recv, zp),
        ):
            rdma = pltpu.make_async_remote_copy(
                src_ref=rblk.at[c],
                dst_ref=dst.at[c],
                send_sem=ss.at[c],
                recv_sem=rs.at[c],
                device_id=peer,
                device_id_type=_MESH_ID,
            )
            rdma.start()
            out.append(rdma)
        return out

    for c in range(N_CH):
        x1_rdmas[c].wait_recv()
        pm_copies[c].wait()
        r_copies[c].wait()
        s = c % 2
        if o_copies[s] is not None:
            o_copies[s].wait()
        yv = pmstage[s] + recvx[c].astype(jnp.float32) + rstage[s]
        ms = jnp.mean(yv * yv, axis=-1, keepdims=True)
        ov = yv * lax.rsqrt(ms + 1e-6) * gamma_ref[...]
        ostage[s] = ov
        ovb = ov.astype(jnp.bfloat16)

        @pl.when(my_x == 0)
        def _():
            rblk[c, 0:HH] = ovb

        @pl.when(my_x == 1)
        def _():
            rblk[c, HH:CH] = ovb

        if c + 2 < N_CH:
            pm_copies.append(
                stage_partial(
                    row0 + (c + 2) * CH + hx, pmstage.at[s], pmin_sem.at[s]
                )
            )
            r_copies.append(
                stage(row0 + (c + 2) * CH + hx, rstage.at[s], rin_sem.at[s])
            )
        x2 = pltpu.make_async_remote_copy(
            src_ref=rblk.at[c, pl.ds(hx, HH)],
            dst_ref=rblk.at[c, pl.ds(hx, HH)],
            send_sem=x2_send.at[c],
            recv_sem=x2_recv.at[c],
            device_id=xp,
            device_id_type=_MESH_ID,
        )
        x2.start()
        x2_rdmas.append(x2)
        oc = pltpu.make_async_copy(
            ostage.at[s],
            out_ref.at[pl.ds(row0 + c * CH + hx, HH), :],
            oout_sem.at[s],
        )
        oc.start()
        o_copies[s] = oc
        if c >= 1:
            x2_rdmas[c - 1].wait_recv()
            g = send_gather(c - 1)
            y_rdmas.append(g[0])
            zo_rdmas.append(g[1])
            consume_peer_half(c - 1)
    x2_rdmas[N_CH - 1].wait_recv()
    g = send_gather(N_CH - 1)
    y_rdmas.append(g[0])
    zo_rdmas.append(g[1])
    consume_peer_half(N_CH - 1)

    c_copies = [None, None]

    def consume(buf, c, dst_row, rows):
        s = c % 2
        if c_copies[s] is not None:
            c_copies[s].wait()
        cstage[s, 0:rows] = buf[c][0:rows].astype(jnp.float32)
        cc = pltpu.make_async_copy(
            cstage.at[s, pl.ds(0, rows)],
            out_ref.at[pl.ds(dst_row, rows), :],
            cout_sem.at[s],
        )
        cc.start()
        c_copies[s] = cc

    zf_rdmas = []
    yf_rdmas = []
    for c in range(N_CH):
        y_rdmas[c].wait_recv()
        zf = pltpu.make_async_remote_copy(
            src_ref=recvy.at[c, pl.ds(0, HH)],
            dst_ref=recvzf.at[c],
            send_sem=zf_send.at[c],
            recv_sem=zf_recv.at[c],
            device_id=zp,
            device_id_type=_MESH_ID,
        )
        zf.start()
        zf_rdmas.append(zf)
        consume(recvy, c, b_y * BLK + c * CH, CH)
        zo_rdmas[c].wait_recv()
        yf = pltpu.make_async_remote_copy(
            src_ref=recvzo.at[c, pl.ds(HH, HH)],
            dst_ref=recvyf.at[c],
            send_sem=yf_send.at[c],
            recv_sem=yf_recv.at[c],
            device_id=yp,
            device_id_type=_MESH_ID,
        )
        yf.start()
        yf_rdmas.append(yf)
        consume(recvzo, c, b_z * BLK + c * CH, CH)
    for c in range(N_CH):
        zf_rdmas[c].wait_recv()
        consume(recvzf, c, b_d * BLK + c * CH, HH)
        yf_rdmas[c].wait_recv()
        consume(recvyf, c, b_d * BLK + c * CH + HH, HH)

    for r in (
        x1_rdmas + x2_rdmas + y_rdmas + zo_rdmas + zf_rdmas + yf_rdmas
    ):
        r.wait_send()
    for oc in o_copies + c_copies + ph_copies:
        if oc is not None:
            oc.wait()


def kernel(partial, resid, gamma):
    return pl.pallas_call(
        _body,
        out_shape=jax.ShapeDtypeStruct((M, D), jnp.float32),
        in_specs=[
            pl.BlockSpec(memory_space=pl.ANY),
            pl.BlockSpec(memory_space=pl.ANY),
            pl.BlockSpec(memory_space=pltpu.MemorySpace.VMEM),
        ],
        out_specs=pl.BlockSpec(memory_space=pl.ANY),
        scratch_shapes=[
            pltpu.VMEM((N_CH, HH, D), jnp.bfloat16),
            pltpu.VMEM((N_CH, HH, D), jnp.bfloat16),
            pltpu.VMEM((N_CH, CH, D), jnp.bfloat16),
            pltpu.VMEM((N_CH, CH, D), jnp.bfloat16),
            pltpu.VMEM((N_CH, CH, D), jnp.bfloat16),
            pltpu.VMEM((N_CH, HH, D), jnp.bfloat16),
            pltpu.VMEM((N_CH, HH, D), jnp.bfloat16),
            pltpu.VMEM((2, HH, D), jnp.float32),
            pltpu.VMEM((2, HH, D), jnp.float32),
            pltpu.VMEM((2, HH, D), jnp.float32),
            pltpu.VMEM((2, HH, D), jnp.float32),
            pltpu.VMEM((2, CH, D), jnp.float32),
            pltpu.VMEM((2, HH, D), jnp.float32),
            pltpu.SemaphoreType.DMA((N_CH,)),
            pltpu.SemaphoreType.DMA((N_CH,)),
            pltpu.SemaphoreType.DMA((N_CH,)),
            pltpu.SemaphoreType.DMA((N_CH,)),
            pltpu.SemaphoreType.DMA((N_CH,)),
            pltpu.SemaphoreType.DMA((N_CH,)),
            pltpu.SemaphoreType.DMA((N_CH,)),
            pltpu.SemaphoreType.DMA((N_CH,)),
            pltpu.SemaphoreType.DMA((N_CH,)),
            pltpu.SemaphoreType.DMA((N_CH,)),
            pltpu.SemaphoreType.DMA((N_CH,)),
            pltpu.SemaphoreType.DMA((N_CH,)),
            pltpu.SemaphoreType.DMA((2,)),
            pltpu.SemaphoreType.DMA((2,)),
            pltpu.SemaphoreType.DMA((2,)),
            pltpu.SemaphoreType.DMA((2,)),
            pltpu.SemaphoreType.DMA((2,)),
            pltpu.SemaphoreType.DMA((2,)),
            pltpu.SemaphoreType.REGULAR((3,)),
        ],
        compiler_params=pltpu.CompilerParams(
            collective_id=0, vmem_limit_bytes=100 * 1024 * 1024
        ),
    )(partial, resid, gamma.reshape(1, -1))
